# baseline (speedup 1.0000x reference)
import functools

import jax
import jax.numpy as jnp
from jax import lax
from jax.experimental import pallas as pl
from jax.experimental.pallas import tpu as pltpu
from jax.experimental.pallas import tpu_sc as plsc

N_NODES = 100000
D = 256
NSEG = 64
NW = 32
SEG_PER_W = NSEG // NW
CHUNK = 192
CB = 4


IDS_SLICE = 6272
IDS_BUF = 6280
BIGI = 1 << 30


def _pool_body(feats_hbm, offs_hbm, out_hbm, offs_v, buf0, buf1, orow,
               sem0, sem1):
    cid = lax.axis_index("c")
    sid = lax.axis_index("s")
    wid = cid * 16 + sid
    pltpu.sync_copy(offs_hbm, offs_v)

    for t in range(SEG_PER_W):
        seg = wid * SEG_PER_W + t
        ov = offs_v[pl.ds(seg, 16)]
        lo = ov[0]
        hi = ov[1]
        lo_al = (lo // 8) * 8
        nch = (hi - lo_al + (CHUNK - 1)) // CHUNK
        npair = (nch + 1) // 2

        def start(c, buf, sem, lo_al=lo_al):
            cs = jnp.minimum(lo_al + c * CHUNK, N_NODES - CHUNK)
            cs = pl.multiple_of(cs, 8)
            pltpu.async_copy(feats_hbm.at[pl.ds(cs, CHUNK)], buf, sem)

        def wait(buf, sem):
            pltpu.make_async_copy(
                feats_hbm.at[pl.ds(0, CHUNK)], buf, sem).wait()

        def compute(buf, c, accs, lo=lo, hi=hi, lo_al=lo_al):
            cs_u = lo_al + c * CHUNK
            cs = jnp.minimum(cs_u, N_NODES - CHUNK)
            a = jnp.maximum(lo, cs_u) - cs
            b = jnp.minimum(hi, cs_u + CHUNK) - cs
            n4 = jnp.maximum(b - a, 0) // 8
            rem_lo = a + 8 * n4
            rem_hi = jnp.maximum(b, rem_lo)

            def step(r, sub, p):
                ss = list(sub[0:4])
                mm = list(sub[4:8])
                nn = list(sub[8:12])
                for j in range(4):
                    v = buf[r, pl.ds(p * 64 + j * 16, 16)]
                    ss[j] = ss[j] + v
                    mm[j] = jnp.maximum(mm[j], v)
                    nn[j] = jnp.minimum(nn[j], v)
                return tuple(ss + mm + nn)

            new = []
            for p in range(CB):
                sub = accs[12 * p:12 * p + 12]

                def quad_body(g, sub, p=p):
                    base = a + 8 * g
                    for u in range(8):
                        sub = step(base + u, sub, p)
                    return sub

                def rem_body(r, sub, p=p):
                    return step(r, sub, p)

                sub = lax.fori_loop(0, n4, quad_body, tuple(sub))
                sub = lax.fori_loop(rem_lo, rem_hi, rem_body, sub)
                new.extend(sub)
            return tuple(new)

        zero = jnp.zeros((16,), jnp.float32)
        ninf = jnp.full((16,), -jnp.inf, jnp.float32)
        pinf = jnp.full((16,), jnp.inf, jnp.float32)
        accs = tuple([zero] * 4 + [ninf] * 4 + [pinf] * 4) * CB

        @pl.when(nch > 0)
        def _():
            start(jnp.int32(0), buf0, sem0)

        def pair_body(pp, accs):
            c0 = 2 * pp
            c1 = c0 + 1

            @pl.when(c1 < nch)
            def _():
                start(c1, buf1, sem1)

            wait(buf0, sem0)
            accs = compute(buf0, c0, accs)

            @pl.when(c0 + 2 < nch)
            def _():
                start(c0 + 2, buf0, sem0)

            @pl.when(c1 < nch)
            def _():
                wait(buf1, sem1)

            accs = compute(buf1, c1, accs)
            return accs

        accs = lax.fori_loop(0, npair, pair_body, accs)

        cnt = jnp.maximum((hi - lo).astype(jnp.float32), 1.0)
        inv = jnp.full((16,), 1.0, jnp.float32) / jnp.full((16,), cnt)
        for p in range(CB):
            sub = accs[12 * p:12 * p + 12]
            for j in range(4):
                col = p * 64 + j * 16
                orow[pl.ds(col, 16)] = sub[j] * inv
                orow[pl.ds(D + col, 16)] = sub[4 + j]
                orow[pl.ds(2 * D + col, 16)] = sub[8 + j]
        dst = pl.multiple_of(seg * (3 * D), 8)
        pltpu.sync_copy(orow, out_hbm.at[pl.ds(dst, 3 * D)])


def _pool(node_feats, offs):
    mesh = plsc.VectorSubcoreMesh(core_axis_name="c", subcore_axis_name="s")
    fn = functools.partial(
        pl.kernel,
        mesh=mesh,
        out_type=jax.ShapeDtypeStruct((NSEG * 3 * D,), jnp.float32),
        scratch_types=[
            pltpu.VMEM((80,), jnp.int32),
            pltpu.VMEM((CHUNK, D), jnp.float32),
            pltpu.VMEM((CHUNK, D), jnp.float32),
            pltpu.VMEM((3 * D,), jnp.float32),
            pltpu.SemaphoreType.DMA,
            pltpu.SemaphoreType.DMA,
        ],
    )(_pool_body)
    return fn(node_feats, offs)


def _mlp_body(p_ref, w1_ref, b1_ref, w2_ref, b2_ref, o_ref):
    h = jnp.dot(p_ref[...], w1_ref[...], preferred_element_type=jnp.float32)
    h = jnp.maximum(h + b1_ref[...], 0.0)
    o_ref[...] = jnp.dot(h, w2_ref[...], preferred_element_type=jnp.float32) + b2_ref[...]


def _mlp(pooled_flat, W1, b1, W2, b2):
    return pl.pallas_call(
        _mlp_body,
        out_shape=jax.ShapeDtypeStruct((NSEG, W2.shape[1]), jnp.float32),
    )(pooled_flat, W1, b1.reshape(1, -1), W2, b2.reshape(1, -1))


def kernel(node_feats, segment_ids, W1, b1, W2, b2):
    ids = segment_ids.astype(jnp.int32)
    offs = jnp.searchsorted(ids, jnp.arange(NSEG + 1, dtype=jnp.int32),
                            method="compare_all").astype(jnp.int32)
    offs = jnp.pad(offs, (0, 80 - (NSEG + 1)), constant_values=N_NODES)
    pooled = _pool(node_feats, offs).reshape(NSEG, 3 * D)
    return _mlp(pooled, W1, b1, W2, b2)


# speedup vs baseline: 1.0283x; 1.0283x over previous
import functools

import jax
import jax.numpy as jnp
from jax import lax
from jax.experimental import pallas as pl
from jax.experimental.pallas import tpu as pltpu
from jax.experimental.pallas import tpu_sc as plsc

N_NODES = 100000
D = 256
NSEG = 64
NW = 32
SEG_PER_W = NSEG // NW
CHUNK = 128
CB = 4


IDS_SLICE = 6272
IDS_BUF = 6280
BIGI = 1 << 30


def _pool_body(feats_hbm, offs_hbm, out_hbm, offs_v, buf0, buf1, orow,
               sem0, sem1):
    cid = lax.axis_index("c")
    sid = lax.axis_index("s")
    wid = cid * 16 + sid
    pltpu.sync_copy(offs_hbm, offs_v)

    for t in range(SEG_PER_W):
        seg = wid * SEG_PER_W + t
        ov = offs_v[pl.ds(seg, 16)]
        lo = ov[0]
        hi = ov[1]
        lo_al = (lo // 8) * 8
        nch = (hi - lo_al + (CHUNK - 1)) // CHUNK
        npair = (nch + 1) // 2

        def start(c, buf, sem, lo_al=lo_al):
            cs = jnp.minimum(lo_al + c * CHUNK, N_NODES - CHUNK)
            cs = pl.multiple_of(cs, 8)
            pltpu.async_copy(feats_hbm.at[pl.ds(cs, CHUNK)], buf, sem)

        def wait(buf, sem):
            pltpu.make_async_copy(
                feats_hbm.at[pl.ds(0, CHUNK)], buf, sem).wait()

        def compute(buf, c, accs, lo=lo, hi=hi, lo_al=lo_al):
            cs_u = lo_al + c * CHUNK
            cs = jnp.minimum(cs_u, N_NODES - CHUNK)
            a = jnp.maximum(lo, cs_u) - cs
            b = jnp.minimum(hi, cs_u + CHUNK) - cs
            n4 = jnp.maximum(b - a, 0) // 4
            rem_lo = a + 4 * n4
            rem_hi = jnp.maximum(b, rem_lo)

            def step(r, sub, p):
                ss = list(sub[0:4])
                mm = list(sub[4:8])
                nn = list(sub[8:12])
                for j in range(4):
                    v = buf[r, pl.ds(p * 64 + j * 16, 16)]
                    ss[j] = ss[j] + v
                    mm[j] = jnp.maximum(mm[j], v)
                    nn[j] = jnp.minimum(nn[j], v)
                return tuple(ss + mm + nn)

            new = []
            for p in range(CB):
                sub = accs[12 * p:12 * p + 12]

                def quad_body(g, sub, p=p):
                    base = a + 4 * g
                    for u in range(4):
                        sub = step(base + u, sub, p)
                    return sub

                def rem_body(r, sub, p=p):
                    return step(r, sub, p)

                sub = lax.fori_loop(0, n4, quad_body, tuple(sub))
                sub = lax.fori_loop(rem_lo, rem_hi, rem_body, sub)
                new.extend(sub)
            return tuple(new)

        zero = jnp.zeros((16,), jnp.float32)
        ninf = jnp.full((16,), -jnp.inf, jnp.float32)
        pinf = jnp.full((16,), jnp.inf, jnp.float32)
        accs = tuple([zero] * 4 + [ninf] * 4 + [pinf] * 4) * CB

        @pl.when(nch > 0)
        def _():
            start(jnp.int32(0), buf0, sem0)

        def pair_body(pp, accs):
            c0 = 2 * pp
            c1 = c0 + 1

            @pl.when(c1 < nch)
            def _():
                start(c1, buf1, sem1)

            wait(buf0, sem0)
            accs = compute(buf0, c0, accs)

            @pl.when(c0 + 2 < nch)
            def _():
                start(c0 + 2, buf0, sem0)

            @pl.when(c1 < nch)
            def _():
                wait(buf1, sem1)

            accs = compute(buf1, c1, accs)
            return accs

        accs = lax.fori_loop(0, npair, pair_body, accs)

        cnt = jnp.maximum((hi - lo).astype(jnp.float32), 1.0)
        inv = jnp.full((16,), 1.0, jnp.float32) / jnp.full((16,), cnt)
        for p in range(CB):
            sub = accs[12 * p:12 * p + 12]
            for j in range(4):
                col = p * 64 + j * 16
                orow[pl.ds(col, 16)] = sub[j] * inv
                orow[pl.ds(D + col, 16)] = sub[4 + j]
                orow[pl.ds(2 * D + col, 16)] = sub[8 + j]
        dst = pl.multiple_of(seg * (3 * D), 8)
        pltpu.sync_copy(orow, out_hbm.at[pl.ds(dst, 3 * D)])


def _pool(node_feats, offs):
    mesh = plsc.VectorSubcoreMesh(core_axis_name="c", subcore_axis_name="s")
    fn = functools.partial(
        pl.kernel,
        mesh=mesh,
        out_type=jax.ShapeDtypeStruct((NSEG * 3 * D,), jnp.float32),
        scratch_types=[
            pltpu.VMEM((80,), jnp.int32),
            pltpu.VMEM((CHUNK, D), jnp.float32),
            pltpu.VMEM((CHUNK, D), jnp.float32),
            pltpu.VMEM((3 * D,), jnp.float32),
            pltpu.SemaphoreType.DMA,
            pltpu.SemaphoreType.DMA,
        ],
    )(_pool_body)
    return fn(node_feats, offs)


def _mlp_body(p_ref, w1_ref, b1_ref, w2_ref, b2_ref, o_ref):
    h = jnp.dot(p_ref[...], w1_ref[...], preferred_element_type=jnp.float32)
    h = jnp.maximum(h + b1_ref[...], 0.0)
    o_ref[...] = jnp.dot(h, w2_ref[...], preferred_element_type=jnp.float32) + b2_ref[...]


def _mlp(pooled_flat, W1, b1, W2, b2):
    return pl.pallas_call(
        _mlp_body,
        out_shape=jax.ShapeDtypeStruct((NSEG, W2.shape[1]), jnp.float32),
    )(pooled_flat, W1, b1.reshape(1, -1), W2, b2.reshape(1, -1))


def kernel(node_feats, segment_ids, W1, b1, W2, b2):
    ids = segment_ids.astype(jnp.int32)
    offs = jnp.searchsorted(ids, jnp.arange(NSEG + 1, dtype=jnp.int32),
                            method="compare_all").astype(jnp.int32)
    offs = jnp.pad(offs, (0, 80 - (NSEG + 1)), constant_values=N_NODES)
    pooled = _pool(node_feats, offs).reshape(NSEG, 3 * D)
    return _mlp(pooled, W1, b1, W2, b2)
